# Initial kernel scaffold; baseline (speedup 1.0000x reference)
import jax
import jax.numpy as jnp
from jax.experimental import pallas as pl


def kernel(x, edge_index, W1, W2):
    raise NotImplementedError("write your pallas kernel here")


# speedup vs baseline: 9.8781x; 9.8781x over previous
import functools

import jax
import jax.numpy as jnp
from jax import lax
from jax.experimental import pallas as pl
from jax.experimental.pallas import tpu as pltpu
from jax.experimental.pallas import tpu_sc as plsc

N = 10000
D = 128
E = 320000
NP = 10240
EP = 323584
CHUNK = 128
CPT = EP // (32 * CHUNK)
EPT = CPT * CHUNK
RPT = NP // 16


def _wid():
    cid = lax.axis_index("c")
    sid = lax.axis_index("s")
    return cid, sid, sid * 2 + cid


def _deg_body(dst_hbm, degp, idxbuf, deg_local):
    cid, sid, wid = _wid()
    zeros16 = jnp.zeros((16,), jnp.float32)
    ones16 = jnp.ones((16,), jnp.float32)

    @pl.loop(0, NP // 16)
    def _(i):
        deg_local[pl.ds(i * 16, 16)] = zeros16

    pltpu.sync_copy(dst_hbm.at[pl.ds(wid * EPT, EPT)], idxbuf)

    @pl.loop(0, EPT // 16)
    def _(j):
        idx = idxbuf[pl.ds(j * 16, 16)]
        plsc.addupdate_scatter(deg_local, [idx], ones16)

    pltpu.sync_copy(deg_local, degp.at[wid])


def _agg_body(xs_hbm, src_hbm, dst_hbm, outp, sidx, didx, rows, acc, sem):
    cid, sid, wid = _wid()
    zeros16 = jnp.zeros((16,), jnp.float32)

    @pl.loop(0, CHUNK)
    def _(i):
        for k in range(D // 16):
            rows[i, pl.ds(k * 16, 16)] = zeros16

    for b in range(RPT // CHUNK):
        pltpu.sync_copy(rows, acc.at[pl.ds(sid * RPT + b * CHUNK, CHUNK)])
    plsc.subcore_barrier()

    @pl.loop(0, CPT)
    def _(i):
        base = wid * EPT + i * CHUNK
        pltpu.sync_copy(src_hbm.at[pl.ds(base, CHUNK)], sidx)
        pltpu.sync_copy(dst_hbm.at[pl.ds(base, CHUNK)], didx)
        pltpu.async_copy(xs_hbm.at[sidx], rows, sem).wait()
        pltpu.sync_copy(rows, acc.at[didx], add=True)

    plsc.subcore_barrier()
    pltpu.sync_copy(acc.at[pl.ds(sid * RPT, RPT)],
                    outp.at[cid].at[pl.ds(sid * RPT, RPT)])


def _make_sc_deg():
    return pl.kernel(
        _deg_body,
        out_type=jax.ShapeDtypeStruct((32, NP), jnp.float32),
        mesh=plsc.VectorSubcoreMesh(core_axis_name="c", subcore_axis_name="s"),
        compiler_params=pltpu.CompilerParams(needs_layout_passes=False),
        scratch_types=[
            pltpu.VMEM((EPT,), jnp.int32),
            pltpu.VMEM((NP,), jnp.float32),
        ],
    )


def _make_sc_agg():
    return pl.kernel(
        _agg_body,
        out_type=jax.ShapeDtypeStruct((2, NP, D), jnp.float32),
        mesh=plsc.VectorSubcoreMesh(core_axis_name="c", subcore_axis_name="s"),
        compiler_params=pltpu.CompilerParams(needs_layout_passes=False),
        scratch_types=[
            pltpu.VMEM((CHUNK,), jnp.int32),
            pltpu.VMEM((CHUNK,), jnp.int32),
            pltpu.VMEM((CHUNK, D), jnp.float32),
            pltpu.VMEM_SHARED((NP, D), jnp.float32),
            pltpu.SemaphoreType.DMA,
        ],
    )


def _inv_col(degc):
    deg = jnp.sum(degc, axis=1, keepdims=True)
    return lax.rsqrt(jnp.maximum(deg, 1.0))


def _prescale_body(x_ref, degc_ref, xs_ref):
    xs_ref[...] = x_ref[...] * _inv_col(degc_ref[...])


def _mid_body(sp_ref, w_ref, degc_ref, hs_ref):
    s = sp_ref[0] + sp_ref[1]
    t = jnp.maximum(jnp.dot(s, w_ref[...], preferred_element_type=jnp.float32), 0.0)
    nrm = jnp.sqrt(jnp.sum(t * t, axis=1, keepdims=True))
    h = t / jnp.maximum(nrm, 1e-12)
    hs_ref[...] = h * _inv_col(degc_ref[...])


def _out_body(sp_ref, w_ref, degc_ref, o_ref):
    s = (sp_ref[0] + sp_ref[1]) * _inv_col(degc_ref[...])
    o_ref[...] = jnp.dot(s, w_ref[...], preferred_element_type=jnp.float32)


def kernel(x, edge_index, W1, W2):
    src = edge_index[0].astype(jnp.int32)
    dst = edge_index[1].astype(jnp.int32)
    pad = jnp.full((EP - E,), N, jnp.int32)
    srcp = jnp.concatenate([src, pad])
    dstp = jnp.concatenate([dst, pad])
    x_pad = jnp.pad(x, ((0, NP - N), (0, 0)))

    degp = _make_sc_deg()(dstp)
    degc = degp.T

    xs = pl.pallas_call(
        _prescale_body,
        out_shape=jax.ShapeDtypeStruct((NP, D), jnp.float32),
    )(x_pad, degc)

    s1 = _make_sc_agg()(xs, srcp, dstp)

    hs = pl.pallas_call(
        _mid_body,
        out_shape=jax.ShapeDtypeStruct((NP, D), jnp.float32),
    )(s1, W1, degc)

    s2 = _make_sc_agg()(hs, srcp, dstp)

    outp = pl.pallas_call(
        _out_body,
        out_shape=jax.ShapeDtypeStruct((NP, D), jnp.float32),
    )(s2, W2, degc)

    return outp[:N]
